# baseline (speedup 1.0000x reference)
import functools

import jax
import jax.numpy as jnp
from jax import lax
from jax.experimental import pallas as pl
from jax.experimental.pallas import tpu as pltpu
from jax.experimental.pallas import tpu_sc as plsc

INP = 2048
HID = 32768
TOPK = 32

BH = 2048
NB = HID // BH

NC = 2
NS = 16
L = 16
NW = NC * NS
CHUNK = 128
NCH = INP // CHUNK


def _enc_body(emb_ref, w_ref, b_ref, out_ref):
    out_ref[...] = lax.dot_general(
        w_ref[...], emb_ref[...],
        (((1,), (0,)), ((), ())),
        preferred_element_type=jnp.float32,
        precision=lax.Precision.HIGHEST,
    ) + b_ref[...]


def _topk_body(enc_ref, idx_ref, val_ref):
    enc = enc_ref[...]
    rows = HID // 128
    iota = (lax.broadcasted_iota(jnp.int32, (rows, 128), 0) * 128
            + lax.broadcasted_iota(jnp.int32, (rows, 128), 1))
    lane = lax.broadcasted_iota(jnp.int32, (TOPK,), 0)
    a0 = jnp.abs(enc)

    def step(t, carry):
        a, idxs, vals = carry
        m = jnp.max(a)
        flat = jnp.min(jnp.where(a == m, iota, jnp.int32(HID)))
        sel = iota == flat
        v = jnp.sum(jnp.where(sel, enc, 0.0))
        idxs = jnp.where(lane == t, flat, idxs)
        vals = jnp.where(lane == t, v, vals)
        a = jnp.where(sel, -1.0, a)
        return a, idxs, vals

    _, idxs, vals = lax.fori_loop(
        0, TOPK, step,
        (a0, jnp.zeros((TOPK,), jnp.int32), jnp.zeros((TOPK,), jnp.float32)))
    idx_ref[...] = idxs
    val_ref[...] = vals


def _dec_body(wd16, idx_hbm, val_hbm, out_hbm,
              rowidx_v, rows_v, col_v, idx_v, val_v, sem):
    wid = lax.axis_index("s") * NC + lax.axis_index("c")
    pltpu.sync_copy(idx_hbm, idx_v)
    pltpu.sync_copy(val_hbm, val_v)

    lanes = lax.broadcasted_iota(jnp.int32, (L,), 0)
    zero = lanes * 0
    wrow = zero + wid // L
    wlane = zero + wid % L
    jvec = plsc.load_gather(idx_v, [wrow, wlane])
    vvec = plsc.load_gather(val_v, [wrow, wlane])

    jrow = jvec // L
    jlane = jvec % L

    for t in range(NCH):
        for s in range(CHUNK // L):
            i_base = t * CHUNK + s * L
            rowidx_v[t, pl.ds(s * L, L)] = (lanes + i_base) * (HID // L) + jrow

    copies = []
    for t in range(NCH):
        c = pltpu.make_async_copy(
            wd16.at[rowidx_v.at[t]],
            rows_v.at[pl.ds(t * CHUNK, CHUNK)],
            sem)
        c.start()
        copies.append(c)
    for c in copies:
        c.wait()

    for t in range(INP // L):
        picked = plsc.load_gather(rows_v, [lanes + t * L, jlane])
        col_v[0, pl.ds(t * L, L)] = picked * vvec

    pltpu.sync_copy(col_v, out_hbm.at[pl.ds(wid, 1)])


@functools.cache
def _dec_gather():
    return pl.kernel(
        _dec_body,
        out_type=jax.ShapeDtypeStruct((NW, INP), jnp.float32),
        mesh=plsc.VectorSubcoreMesh(
            core_axis_name="c", subcore_axis_name="s",
            num_cores=NC, num_subcores=NS),
        compiler_params=pltpu.CompilerParams(
            needs_layout_passes=False, use_tc_tiling_on_sc=False),
        scratch_types=[
            pltpu.VMEM((NCH, CHUNK), jnp.int32),
            pltpu.VMEM((INP, L), jnp.float32),
            pltpu.VMEM((1, INP), jnp.float32),
            pltpu.VMEM((TOPK // L, L), jnp.int32),
            pltpu.VMEM((TOPK // L, L), jnp.float32),
            pltpu.SemaphoreType.DMA,
        ],
    )


def _fin_body(cols_ref, b_ref, out_ref):
    out_ref[...] = jnp.sum(cols_ref[...], axis=0, keepdims=True) + b_ref[...]


def kernel(emb, W_enc, b_enc, W_dec, b_dec):
    emb2 = emb.reshape(INP, 1)
    benc2 = b_enc.reshape(HID, 1)

    enc2 = pl.pallas_call(
        _enc_body,
        grid=(NB,),
        in_specs=[
            pl.BlockSpec((INP, 1), lambda i: (0, 0)),
            pl.BlockSpec((BH, INP), lambda i: (i, 0)),
            pl.BlockSpec((BH, 1), lambda i: (i, 0)),
        ],
        out_specs=pl.BlockSpec((BH, 1), lambda i: (i, 0)),
        out_shape=jax.ShapeDtypeStruct((HID, 1), jnp.float32),
        compiler_params=pltpu.CompilerParams(vmem_limit_bytes=100 * 2**20),
    )(emb2, W_enc, benc2)

    idx, val = pl.pallas_call(
        _topk_body,
        out_shape=[
            jax.ShapeDtypeStruct((TOPK,), jnp.int32),
            jax.ShapeDtypeStruct((TOPK,), jnp.float32),
        ],
    )(enc2.reshape(HID // 128, 128))

    wd16 = W_dec.reshape(INP * HID // L, L)
    cols = _dec_gather()(wd16, idx.reshape(TOPK // L, L), val.reshape(TOPK // L, L))

    recon2 = pl.pallas_call(
        _fin_body,
        out_shape=jax.ShapeDtypeStruct((1, INP), jnp.float32),
    )(cols, b_dec.reshape(1, INP))

    return enc2.reshape(HID), recon2.reshape(INP)


# speedup vs baseline: 2.2091x; 2.2091x over previous
import functools

import jax
import jax.numpy as jnp
from jax import lax
from jax.experimental import pallas as pl
from jax.experimental.pallas import tpu as pltpu
from jax.experimental.pallas import tpu_sc as plsc

INP = 2048
HID = 32768
TOPK = 32

BH = 2048
NB = HID // BH

NC = 2
NS = 16
L = 16
NW = NC * NS
ROWC = 256
NCH = INP // ROWC


def _enc_body(emb_ref, w_ref, b_ref, out_ref):
    out_ref[...] = lax.dot_general(
        w_ref[...], emb_ref[...],
        (((1,), (0,)), ((), ())),
        preferred_element_type=jnp.float32,
        precision=lax.Precision.DEFAULT,
    ) + b_ref[...]


def _topk_body(enc_ref, idx_ref, val_ref):
    enc = enc_ref[...]
    iota = (lax.broadcasted_iota(jnp.int32, (NB, BH), 0) * BH
            + lax.broadcasted_iota(jnp.int32, (NB, BH), 1))
    lane = lax.broadcasted_iota(jnp.int32, (TOPK,), 0)
    a0 = jnp.abs(enc)

    def step(t, carry):
        a, idxs, vals = carry
        m = jnp.max(a)
        flat = jnp.min(jnp.where(a == m, iota, jnp.int32(HID)))
        sel = iota == flat
        v = jnp.sum(jnp.where(sel, enc, 0.0))
        idxs = jnp.where(lane == t, flat, idxs)
        vals = jnp.where(lane == t, v, vals)
        a = jnp.where(sel, -1.0, a)
        return a, idxs, vals

    _, idxs, vals = lax.fori_loop(
        0, TOPK, step,
        (a0, jnp.zeros((TOPK,), jnp.int32), jnp.zeros((TOPK,), jnp.float32)))
    idx_ref[...] = idxs
    val_ref[...] = vals


def _dec_body(wd, idx_hbm, val_hbm, out_hbm,
              rows_a, rows_b, col_v, idx_v, val_v, sem_a, sem_b):
    wid = lax.axis_index("s") * NC + lax.axis_index("c")
    pltpu.sync_copy(idx_hbm, idx_v)
    pltpu.sync_copy(val_hbm, val_v)

    lanes = lax.broadcasted_iota(jnp.int32, (L,), 0)
    zero = lanes * 0
    i0 = idx_v[pl.ds(0, L)]
    i1 = idx_v[pl.ds(L, L)]
    j = jnp.max(jnp.maximum(jnp.where(lanes == wid, i0, -1),
                            jnp.where(lanes + L == wid, i1, -1)))
    v0 = val_v[pl.ds(0, L)]
    v1 = val_v[pl.ds(L, L)]
    v = jnp.sum(jnp.where(lanes == wid, v0, 0.0)
                + jnp.where(lanes + L == wid, v1, 0.0))

    c128 = j // 128
    jlane = zero + j % 128

    rows = (rows_a, rows_b)
    sems = (sem_a, sem_b)

    def fetch(t, buf):
        return pltpu.make_async_copy(
            wd.at[pl.ds(t * ROWC, ROWC), pl.ds(c128 * 128, 128)],
            rows[buf],
            sems[buf])

    cp = fetch(0, 0)
    cp.start()
    for t in range(NCH):
        nxt = None
        if t + 1 < NCH:
            nxt = fetch(t + 1, (t + 1) % 2)
            nxt.start()
        cp.wait()
        buf = rows[t % 2]
        for s in range(ROWC // L):
            p0 = t * ROWC + s * L
            picked = plsc.load_gather(buf, [lanes + s * L, jlane])
            col_v[p0 // 128, pl.ds(p0 % 128, L)] = picked * v
        cp = nxt

    pltpu.sync_copy(col_v, out_hbm.at[wid])


@functools.cache
def _dec_gather():
    return pl.kernel(
        _dec_body,
        out_type=jax.ShapeDtypeStruct((NW, INP // 128, 128), jnp.float32),
        mesh=plsc.VectorSubcoreMesh(
            core_axis_name="c", subcore_axis_name="s",
            num_cores=NC, num_subcores=NS),
        compiler_params=pltpu.CompilerParams(
            needs_layout_passes=False, use_tc_tiling_on_sc=True),
        scratch_types=[
            pltpu.VMEM((ROWC, 128), jnp.float32),
            pltpu.VMEM((ROWC, 128), jnp.float32),
            pltpu.VMEM((INP // 128, 128), jnp.float32),
            pltpu.VMEM((TOPK,), jnp.int32),
            pltpu.VMEM((TOPK,), jnp.float32),
            pltpu.SemaphoreType.DMA,
            pltpu.SemaphoreType.DMA,
        ],
    )


def _fin_body(cols_ref, b_ref, out_ref):
    out_ref[...] = jnp.sum(cols_ref[...], axis=0) + b_ref[...]


def kernel(emb, W_enc, b_enc, W_dec, b_dec):
    emb2 = emb.reshape(INP, 1)
    benc2 = b_enc.reshape(HID, 1)

    enc2 = pl.pallas_call(
        _enc_body,
        grid=(NB,),
        in_specs=[
            pl.BlockSpec((INP, 1), lambda i: (0, 0)),
            pl.BlockSpec((BH, INP), lambda i: (i, 0)),
            pl.BlockSpec((BH, 1), lambda i: (i, 0)),
        ],
        out_specs=pl.BlockSpec((BH, 1), lambda i: (i, 0)),
        out_shape=jax.ShapeDtypeStruct((HID, 1), jnp.float32),
        compiler_params=pltpu.CompilerParams(vmem_limit_bytes=100 * 2**20),
    )(emb2, W_enc, benc2)

    idx, val = pl.pallas_call(
        _topk_body,
        out_shape=[
            jax.ShapeDtypeStruct((TOPK,), jnp.int32),
            jax.ShapeDtypeStruct((TOPK,), jnp.float32),
        ],
    )(enc2.reshape(NB, BH))

    cols = _dec_gather()(W_dec, idx, val)

    recon2 = pl.pallas_call(
        _fin_body,
        out_shape=jax.ShapeDtypeStruct((INP // 128, 128), jnp.float32),
    )(cols, b_dec.reshape(INP // 128, 128))

    return enc2.reshape(HID), recon2.reshape(INP)


# speedup vs baseline: 2.7024x; 1.2233x over previous
import functools

import jax
import jax.numpy as jnp
from jax import lax
from jax.experimental import pallas as pl
from jax.experimental.pallas import tpu as pltpu
from jax.experimental.pallas import tpu_sc as plsc

INP = 2048
HID = 32768
TOPK = 32

BH = 2048
NB = HID // BH

NC = 2
NS = 16
L = 16
NW = NC * NS
ROWC = 256
NCH = INP // ROWC


def _enc_body(emb_ref, w_ref, b_ref, out_ref):
    r = lax.dot_general(
        emb_ref[...], w_ref[...],
        (((1,), (1,)), ((), ())),
        preferred_element_type=jnp.float32,
        precision=lax.Precision.DEFAULT,
    )
    out_ref[...] = r.reshape(1, 1, BH) + b_ref[...]


def _topk_body(enc_ref, idx_ref, val_ref):
    enc = enc_ref[...].reshape(NB, BH)
    iota = (lax.broadcasted_iota(jnp.int32, (NB, BH), 0) * BH
            + lax.broadcasted_iota(jnp.int32, (NB, BH), 1))
    lane = lax.broadcasted_iota(jnp.int32, (TOPK,), 0)
    a0 = jnp.abs(enc)

    def step(t, carry):
        a, idxs, vals = carry
        m = jnp.max(a)
        flat = jnp.min(jnp.where(a == m, iota, jnp.int32(HID)))
        sel = iota == flat
        v = jnp.sum(jnp.where(sel, enc, 0.0))
        idxs = jnp.where(lane == t, flat, idxs)
        vals = jnp.where(lane == t, v, vals)
        a = jnp.where(sel, -1.0, a)
        return a, idxs, vals

    _, idxs, vals = lax.fori_loop(
        0, TOPK, step,
        (a0, jnp.zeros((TOPK,), jnp.int32), jnp.zeros((TOPK,), jnp.float32)))
    idx_ref[...] = idxs
    val_ref[...] = vals


def _dec_body(wd, idx_hbm, val_hbm, out_hbm,
              rows_a, rows_b, col_v, idx_v, val_v, sem_a, sem_b):
    wid = lax.axis_index("s") * NC + lax.axis_index("c")
    pltpu.sync_copy(idx_hbm, idx_v)
    pltpu.sync_copy(val_hbm, val_v)

    lanes = lax.broadcasted_iota(jnp.int32, (L,), 0)
    zero = lanes * 0
    i0 = idx_v[pl.ds(0, L)]
    i1 = idx_v[pl.ds(L, L)]
    j = jnp.max(jnp.maximum(jnp.where(lanes == wid, i0, -1),
                            jnp.where(lanes + L == wid, i1, -1)))
    v0 = val_v[pl.ds(0, L)]
    v1 = val_v[pl.ds(L, L)]
    v = jnp.sum(jnp.where(lanes == wid, v0, 0.0)
                + jnp.where(lanes + L == wid, v1, 0.0))

    c128 = j // 128
    jlane = zero + j % 128

    rows = (rows_a, rows_b)
    sems = (sem_a, sem_b)

    def fetch(t, buf):
        return pltpu.make_async_copy(
            wd.at[pl.ds(t * ROWC, ROWC), pl.ds(c128 * 128, 128)],
            rows[buf],
            sems[buf])

    cp = fetch(0, 0)
    cp.start()
    for t in range(NCH):
        nxt = None
        if t + 1 < NCH:
            nxt = fetch(t + 1, (t + 1) % 2)
            nxt.start()
        cp.wait()
        buf = rows[t % 2]
        for s in range(ROWC // L):
            p0 = t * ROWC + s * L
            picked = plsc.load_gather(buf, [lanes + s * L, jlane])
            col_v[p0 // 128, pl.ds(p0 % 128, L)] = picked * v
        cp = nxt

    pltpu.sync_copy(col_v, out_hbm.at[wid])


@functools.cache
def _dec_gather():
    return pl.kernel(
        _dec_body,
        out_type=jax.ShapeDtypeStruct((NW, INP // 128, 128), jnp.float32),
        mesh=plsc.VectorSubcoreMesh(
            core_axis_name="c", subcore_axis_name="s",
            num_cores=NC, num_subcores=NS),
        compiler_params=pltpu.CompilerParams(
            needs_layout_passes=False, use_tc_tiling_on_sc=True),
        scratch_types=[
            pltpu.VMEM((ROWC, 128), jnp.float32),
            pltpu.VMEM((ROWC, 128), jnp.float32),
            pltpu.VMEM((INP // 128, 128), jnp.float32),
            pltpu.VMEM((TOPK,), jnp.int32),
            pltpu.VMEM((TOPK,), jnp.float32),
            pltpu.SemaphoreType.DMA,
            pltpu.SemaphoreType.DMA,
        ],
    )


def _fin_body(cols_ref, b_ref, out_ref):
    out_ref[...] = jnp.sum(cols_ref[...], axis=0) + b_ref[...]


def kernel(emb, W_enc, b_enc, W_dec, b_dec):
    emb2 = emb.reshape(1, INP)
    benc2 = b_enc.reshape(NB, 1, BH)

    enc2 = pl.pallas_call(
        _enc_body,
        grid=(NB,),
        in_specs=[
            pl.BlockSpec((1, INP), lambda i: (0, 0)),
            pl.BlockSpec((BH, INP), lambda i: (i, 0)),
            pl.BlockSpec((1, 1, BH), lambda i: (i, 0, 0)),
        ],
        out_specs=pl.BlockSpec((1, 1, BH), lambda i: (i, 0, 0)),
        out_shape=jax.ShapeDtypeStruct((NB, 1, BH), jnp.float32),
        compiler_params=pltpu.CompilerParams(vmem_limit_bytes=100 * 2**20),
    )(emb2, W_enc, benc2)

    idx, val = pl.pallas_call(
        _topk_body,
        out_shape=[
            jax.ShapeDtypeStruct((TOPK,), jnp.int32),
            jax.ShapeDtypeStruct((TOPK,), jnp.float32),
        ],
    )(enc2)

    cols = _dec_gather()(W_dec, idx, val)

    recon2 = pl.pallas_call(
        _fin_body,
        out_shape=jax.ShapeDtypeStruct((INP // 128, 128), jnp.float32),
    )(cols, b_dec.reshape(INP // 128, 128))

    return enc2.reshape(HID), recon2.reshape(INP)


# speedup vs baseline: 2.7408x; 1.0142x over previous
import functools

import jax
import jax.numpy as jnp
from jax import lax
from jax.experimental import pallas as pl
from jax.experimental.pallas import tpu as pltpu
from jax.experimental.pallas import tpu_sc as plsc

INP = 2048
HID = 32768
TOPK = 32

BH = 2048
NB = HID // BH

NC = 2
NS = 16
L = 16
NW = NC * NS
ROWC = 256
NCH = INP // ROWC


def _enc_body(emb_ref, w_ref, b_ref, out_ref):
    r = lax.dot_general(
        emb_ref[...], w_ref[...],
        (((1,), (1,)), ((), ())),
        preferred_element_type=jnp.float32,
        precision=lax.Precision.DEFAULT,
    )
    out_ref[...] = r.reshape(1, 1, BH) + b_ref[...]


def _topk_body(enc_ref, idx_ref, val_ref, a_sc):
    enc = enc_ref[...].reshape(NB, BH)
    a_sc[...] = jnp.abs(enc)
    rows_i = lax.broadcasted_iota(jnp.int32, (NB, 1), 0)
    lanes_i = lax.broadcasted_iota(jnp.int32, (1, BH), 1)
    lane32 = lax.broadcasted_iota(jnp.int32, (TOPK,), 0)
    rowmax0 = jnp.max(a_sc[...], axis=1, keepdims=True)

    def step(t, carry):
        rowmax, idxs, vals = carry
        m = jnp.max(rowmax)
        row = jnp.min(jnp.where(rowmax == m, rows_i, NB))
        arow = a_sc[pl.ds(row, 1), :]
        erow = enc_ref[pl.ds(row, 1), :, :].reshape(1, BH)
        lane = jnp.min(jnp.where(arow == m, lanes_i, BH))
        flat = row * BH + lane
        v = jnp.sum(jnp.where(lanes_i == lane, erow, 0.0))
        arow2 = jnp.where(lanes_i == lane, -1.0, arow)
        a_sc[pl.ds(row, 1), :] = arow2
        rowmax = jnp.where(rows_i == row, jnp.max(arow2), rowmax)
        idxs = jnp.where(lane32 == t, flat, idxs)
        vals = jnp.where(lane32 == t, v, vals)
        return rowmax, idxs, vals

    _, idxs, vals = lax.fori_loop(
        0, TOPK, step,
        (rowmax0, jnp.zeros((TOPK,), jnp.int32), jnp.zeros((TOPK,), jnp.float32)))
    idx_ref[...] = idxs
    val_ref[...] = vals


def _dec_body(wd, idx_hbm, val_hbm, out_hbm,
              rows_a, rows_b, col_v, idx_v, val_v, sem_a, sem_b):
    wid = lax.axis_index("s") * NC + lax.axis_index("c")
    pltpu.sync_copy(idx_hbm, idx_v)
    pltpu.sync_copy(val_hbm, val_v)

    lanes = lax.broadcasted_iota(jnp.int32, (L,), 0)
    zero = lanes * 0
    i0 = idx_v[pl.ds(0, L)]
    i1 = idx_v[pl.ds(L, L)]
    j = jnp.max(jnp.maximum(jnp.where(lanes == wid, i0, -1),
                            jnp.where(lanes + L == wid, i1, -1)))
    v0 = val_v[pl.ds(0, L)]
    v1 = val_v[pl.ds(L, L)]
    v = jnp.sum(jnp.where(lanes == wid, v0, 0.0)
                + jnp.where(lanes + L == wid, v1, 0.0))

    c128 = j // 128
    jlane = zero + j % 128

    rows = (rows_a, rows_b)
    sems = (sem_a, sem_b)

    def fetch(t, buf):
        return pltpu.make_async_copy(
            wd.at[pl.ds(t * ROWC, ROWC), pl.ds(c128 * 128, 128)],
            rows[buf],
            sems[buf])

    cp = fetch(0, 0)
    cp.start()
    for t in range(NCH):
        nxt = None
        if t + 1 < NCH:
            nxt = fetch(t + 1, (t + 1) % 2)
            nxt.start()
        cp.wait()
        buf = rows[t % 2]
        for s in range(ROWC // L):
            p0 = t * ROWC + s * L
            picked = plsc.load_gather(buf, [lanes + s * L, jlane])
            col_v[p0 // 128, pl.ds(p0 % 128, L)] = picked * v
        cp = nxt

    pltpu.sync_copy(col_v, out_hbm.at[wid])


@functools.cache
def _dec_gather():
    return pl.kernel(
        _dec_body,
        out_type=jax.ShapeDtypeStruct((NW, INP // 128, 128), jnp.float32),
        mesh=plsc.VectorSubcoreMesh(
            core_axis_name="c", subcore_axis_name="s",
            num_cores=NC, num_subcores=NS),
        compiler_params=pltpu.CompilerParams(
            needs_layout_passes=False, use_tc_tiling_on_sc=True),
        scratch_types=[
            pltpu.VMEM((ROWC, 128), jnp.float32),
            pltpu.VMEM((ROWC, 128), jnp.float32),
            pltpu.VMEM((INP // 128, 128), jnp.float32),
            pltpu.VMEM((TOPK,), jnp.int32),
            pltpu.VMEM((TOPK,), jnp.float32),
            pltpu.SemaphoreType.DMA,
            pltpu.SemaphoreType.DMA,
        ],
    )


def _fin_body(cols_ref, b_ref, out_ref):
    out_ref[...] = jnp.sum(cols_ref[...], axis=0) + b_ref[...]


def kernel(emb, W_enc, b_enc, W_dec, b_dec):
    emb2 = emb.reshape(1, INP)
    benc2 = b_enc.reshape(NB, 1, BH)

    enc2 = pl.pallas_call(
        _enc_body,
        grid=(NB,),
        in_specs=[
            pl.BlockSpec((1, INP), lambda i: (0, 0)),
            pl.BlockSpec((BH, INP), lambda i: (i, 0)),
            pl.BlockSpec((1, 1, BH), lambda i: (i, 0, 0)),
        ],
        out_specs=pl.BlockSpec((1, 1, BH), lambda i: (i, 0, 0)),
        out_shape=jax.ShapeDtypeStruct((NB, 1, BH), jnp.float32),
        compiler_params=pltpu.CompilerParams(vmem_limit_bytes=100 * 2**20),
    )(emb2, W_enc, benc2)

    idx, val = pl.pallas_call(
        _topk_body,
        out_shape=[
            jax.ShapeDtypeStruct((TOPK,), jnp.int32),
            jax.ShapeDtypeStruct((TOPK,), jnp.float32),
        ],
        scratch_shapes=[pltpu.VMEM((NB, BH), jnp.float32)],
    )(enc2)

    cols = _dec_gather()(W_dec, idx, val)

    recon2 = pl.pallas_call(
        _fin_body,
        out_shape=jax.ShapeDtypeStruct((INP // 128, 128), jnp.float32),
    )(cols, b_dec.reshape(INP // 128, 128))

    return enc2.reshape(HID), recon2.reshape(INP)


# speedup vs baseline: 2.7735x; 1.0119x over previous
import functools

import jax
import jax.numpy as jnp
from jax import lax
from jax.experimental import pallas as pl
from jax.experimental.pallas import tpu as pltpu
from jax.experimental.pallas import tpu_sc as plsc

INP = 2048
HID = 32768
TOPK = 32

BH = 2048
NB = HID // BH

NC = 2
NS = 16
L = 16
NW = NC * NS
ROWC = 256
NCH = INP // ROWC


def _enc_body(emb_ref, w_ref, b_ref, enc_ref, idx_ref, val_ref, a_sc):
    i = pl.program_id(0)
    r = lax.dot_general(
        emb_ref[...], w_ref[...],
        (((1,), (1,)), ((), ())),
        preferred_element_type=jnp.float32,
        precision=lax.Precision.DEFAULT,
    )
    r3 = r.reshape(1, 1, BH) + b_ref[...]
    enc_ref[pl.ds(i, 1), :, :] = r3
    a_sc[pl.ds(i, 1), :] = jnp.abs(r3).reshape(1, BH)

    @pl.when(i == NB - 1)
    def _():
        rows_i = lax.broadcasted_iota(jnp.int32, (NB, 1), 0)
        lanes_i = lax.broadcasted_iota(jnp.int32, (1, BH), 1)
        lane32 = lax.broadcasted_iota(jnp.int32, (TOPK,), 0)
        rowmax0 = jnp.max(a_sc[...], axis=1, keepdims=True)

        def step(t, carry):
            rowmax, idxs, vals = carry
            m = jnp.max(rowmax)
            row = jnp.min(jnp.where(rowmax == m, rows_i, NB))
            arow = a_sc[pl.ds(row, 1), :]
            erow = enc_ref[pl.ds(row, 1), :, :].reshape(1, BH)
            lane = jnp.min(jnp.where(arow == m, lanes_i, BH))
            flat = row * BH + lane
            v = jnp.sum(jnp.where(lanes_i == lane, erow, 0.0))
            arow2 = jnp.where(lanes_i == lane, -1.0, arow)
            a_sc[pl.ds(row, 1), :] = arow2
            rowmax = jnp.where(rows_i == row, jnp.max(arow2), rowmax)
            idxs = jnp.where(lane32 == t, flat, idxs)
            vals = jnp.where(lane32 == t, v, vals)
            return rowmax, idxs, vals

        _, idxs, vals = lax.fori_loop(
            0, TOPK, step,
            (rowmax0, jnp.zeros((TOPK,), jnp.int32),
             jnp.zeros((TOPK,), jnp.float32)))
        idx_ref[...] = idxs
        val_ref[...] = vals


def _dec_body(wd, idx_hbm, val_hbm, out_hbm,
              rows_a, rows_b, col_v, idx_v, val_v, sem_a, sem_b):
    wid = lax.axis_index("s") * NC + lax.axis_index("c")
    pltpu.sync_copy(idx_hbm, idx_v)
    pltpu.sync_copy(val_hbm, val_v)

    lanes = lax.broadcasted_iota(jnp.int32, (L,), 0)
    zero = lanes * 0
    i0 = idx_v[pl.ds(0, L)]
    i1 = idx_v[pl.ds(L, L)]
    j = jnp.max(jnp.maximum(jnp.where(lanes == wid, i0, -1),
                            jnp.where(lanes + L == wid, i1, -1)))
    v0 = val_v[pl.ds(0, L)]
    v1 = val_v[pl.ds(L, L)]
    v = jnp.sum(jnp.where(lanes == wid, v0, 0.0)
                + jnp.where(lanes + L == wid, v1, 0.0))

    c128 = j // 128
    jlane = zero + j % 128

    rows = (rows_a, rows_b)
    sems = (sem_a, sem_b)

    def fetch(t, buf):
        return pltpu.make_async_copy(
            wd.at[pl.ds(t * ROWC, ROWC), pl.ds(c128 * 128, 128)],
            rows[buf],
            sems[buf])

    cp = fetch(0, 0)
    cp.start()
    for t in range(NCH):
        nxt = None
        if t + 1 < NCH:
            nxt = fetch(t + 1, (t + 1) % 2)
            nxt.start()
        cp.wait()
        buf = rows[t % 2]
        for s in range(ROWC // L):
            p0 = t * ROWC + s * L
            picked = plsc.load_gather(buf, [lanes + s * L, jlane])
            col_v[p0 // 128, pl.ds(p0 % 128, L)] = picked * v
        cp = nxt

    pltpu.sync_copy(col_v, out_hbm.at[wid])


@functools.cache
def _dec_gather():
    return pl.kernel(
        _dec_body,
        out_type=jax.ShapeDtypeStruct((NW, INP // 128, 128), jnp.float32),
        mesh=plsc.VectorSubcoreMesh(
            core_axis_name="c", subcore_axis_name="s",
            num_cores=NC, num_subcores=NS),
        compiler_params=pltpu.CompilerParams(
            needs_layout_passes=False, use_tc_tiling_on_sc=True),
        scratch_types=[
            pltpu.VMEM((ROWC, 128), jnp.float32),
            pltpu.VMEM((ROWC, 128), jnp.float32),
            pltpu.VMEM((INP // 128, 128), jnp.float32),
            pltpu.VMEM((TOPK,), jnp.int32),
            pltpu.VMEM((TOPK,), jnp.float32),
            pltpu.SemaphoreType.DMA,
            pltpu.SemaphoreType.DMA,
        ],
    )


def _fin_body(cols_ref, b_ref, out_ref):
    out_ref[...] = jnp.sum(cols_ref[...], axis=0) + b_ref[...]


def kernel(emb, W_enc, b_enc, W_dec, b_dec):
    emb2 = emb.reshape(1, INP)
    benc2 = b_enc.reshape(NB, 1, BH)

    enc2, idx, val = pl.pallas_call(
        _enc_body,
        grid=(NB,),
        in_specs=[
            pl.BlockSpec((1, INP), lambda i: (0, 0)),
            pl.BlockSpec((BH, INP), lambda i: (i, 0)),
            pl.BlockSpec((1, 1, BH), lambda i: (i, 0, 0)),
        ],
        out_specs=[
            pl.BlockSpec((NB, 1, BH), lambda i: (0, 0, 0)),
            pl.BlockSpec((TOPK,), lambda i: (0,)),
            pl.BlockSpec((TOPK,), lambda i: (0,)),
        ],
        out_shape=[
            jax.ShapeDtypeStruct((NB, 1, BH), jnp.float32),
            jax.ShapeDtypeStruct((TOPK,), jnp.int32),
            jax.ShapeDtypeStruct((TOPK,), jnp.float32),
        ],
        scratch_shapes=[pltpu.VMEM((NB, BH), jnp.float32)],
        compiler_params=pltpu.CompilerParams(vmem_limit_bytes=100 * 2**20),
    )(emb2, W_enc, benc2)

    cols = _dec_gather()(W_dec, idx, val)

    recon2 = pl.pallas_call(
        _fin_body,
        out_shape=jax.ShapeDtypeStruct((INP // 128, 128), jnp.float32),
    )(cols, b_dec.reshape(INP // 128, 128))

    return enc2.reshape(HID), recon2.reshape(INP)


# speedup vs baseline: 2.8789x; 1.0380x over previous
import functools

import jax
import jax.numpy as jnp
from jax import lax
from jax.experimental import pallas as pl
from jax.experimental.pallas import tpu as pltpu
from jax.experimental.pallas import tpu_sc as plsc

INP = 2048
HID = 32768
TOPK = 32

BH = 1024
NB = HID // BH

NC = 2
NS = 16
L = 16
NW = NC * NS
ROWC = 256
NCH = INP // ROWC


def _enc_body(emb_ref, w_ref, b_ref, enc_ref, idx_ref, val_ref, a_sc):
    i = pl.program_id(0)
    r = lax.dot_general(
        emb_ref[...], w_ref[...],
        (((1,), (1,)), ((), ())),
        preferred_element_type=jnp.float32,
        precision=lax.Precision.DEFAULT,
    )
    r3 = r.reshape(1, 1, BH) + b_ref[...]
    enc_ref[pl.ds(i, 1), :, :] = r3
    a_sc[pl.ds(i, 1), :] = jnp.abs(r3).reshape(1, BH)

    @pl.when(i == NB - 1)
    def _():
        rows_i = lax.broadcasted_iota(jnp.int32, (NB, 1), 0)
        lanes_i = lax.broadcasted_iota(jnp.int32, (1, BH), 1)
        lane32 = lax.broadcasted_iota(jnp.int32, (TOPK,), 0)
        rowmax0 = jnp.max(a_sc[...], axis=1, keepdims=True)

        def step(t, carry):
            rowmax, idxs, vals = carry
            m = jnp.max(rowmax)
            row = jnp.min(jnp.where(rowmax == m, rows_i, NB))
            arow = a_sc[pl.ds(row, 1), :]
            erow = enc_ref[pl.ds(row, 1), :, :].reshape(1, BH)
            lane = jnp.min(jnp.where(arow == m, lanes_i, BH))
            flat = row * BH + lane
            v = jnp.sum(jnp.where(lanes_i == lane, erow, 0.0))
            arow2 = jnp.where(lanes_i == lane, -1.0, arow)
            a_sc[pl.ds(row, 1), :] = arow2
            rowmax = jnp.where(rows_i == row, jnp.max(arow2), rowmax)
            idxs = jnp.where(lane32 == t, flat, idxs)
            vals = jnp.where(lane32 == t, v, vals)
            return rowmax, idxs, vals

        _, idxs, vals = lax.fori_loop(
            0, TOPK, step,
            (rowmax0, jnp.zeros((TOPK,), jnp.int32),
             jnp.zeros((TOPK,), jnp.float32)))
        idx_ref[...] = idxs
        val_ref[...] = vals


def _dec_body(wd, idx_hbm, val_hbm, out_hbm,
              rows_a, rows_b, col_v, idx_v, val_v, sem_a, sem_b):
    wid = lax.axis_index("s") * NC + lax.axis_index("c")
    pltpu.sync_copy(idx_hbm, idx_v)
    pltpu.sync_copy(val_hbm, val_v)

    lanes = lax.broadcasted_iota(jnp.int32, (L,), 0)
    zero = lanes * 0
    i0 = idx_v[pl.ds(0, L)]
    i1 = idx_v[pl.ds(L, L)]
    j = jnp.max(jnp.maximum(jnp.where(lanes == wid, i0, -1),
                            jnp.where(lanes + L == wid, i1, -1)))
    v0 = val_v[pl.ds(0, L)]
    v1 = val_v[pl.ds(L, L)]
    v = jnp.sum(jnp.where(lanes == wid, v0, 0.0)
                + jnp.where(lanes + L == wid, v1, 0.0))

    c128 = j // 128
    jlane = zero + j % 128

    rows = (rows_a, rows_b)
    sems = (sem_a, sem_b)

    def fetch(t, buf):
        return pltpu.make_async_copy(
            wd.at[pl.ds(t * ROWC, ROWC), pl.ds(c128 * 128, 128)],
            rows[buf],
            sems[buf])

    cp = fetch(0, 0)
    cp.start()
    for t in range(NCH):
        nxt = None
        if t + 1 < NCH:
            nxt = fetch(t + 1, (t + 1) % 2)
            nxt.start()
        cp.wait()
        buf = rows[t % 2]
        for s in range(ROWC // L):
            p0 = t * ROWC + s * L
            picked = plsc.load_gather(buf, [lanes + s * L, jlane])
            col_v[p0 // 128, pl.ds(p0 % 128, L)] = picked * v
        cp = nxt

    pltpu.sync_copy(col_v, out_hbm.at[wid])


@functools.cache
def _dec_gather():
    return pl.kernel(
        _dec_body,
        out_type=jax.ShapeDtypeStruct((NW, INP // 128, 128), jnp.float32),
        mesh=plsc.VectorSubcoreMesh(
            core_axis_name="c", subcore_axis_name="s",
            num_cores=NC, num_subcores=NS),
        compiler_params=pltpu.CompilerParams(
            needs_layout_passes=False, use_tc_tiling_on_sc=True),
        scratch_types=[
            pltpu.VMEM((ROWC, 128), jnp.float32),
            pltpu.VMEM((ROWC, 128), jnp.float32),
            pltpu.VMEM((INP // 128, 128), jnp.float32),
            pltpu.VMEM((TOPK,), jnp.int32),
            pltpu.VMEM((TOPK,), jnp.float32),
            pltpu.SemaphoreType.DMA,
            pltpu.SemaphoreType.DMA,
        ],
    )


def _fin_body(cols_ref, b_ref, out_ref):
    out_ref[...] = jnp.sum(cols_ref[...], axis=0) + b_ref[...]


def kernel(emb, W_enc, b_enc, W_dec, b_dec):
    emb2 = emb.reshape(1, INP)
    benc2 = b_enc.reshape(NB, 1, BH)

    enc2, idx, val = pl.pallas_call(
        _enc_body,
        grid=(NB,),
        in_specs=[
            pl.BlockSpec((1, INP), lambda i: (0, 0)),
            pl.BlockSpec((BH, INP), lambda i: (i, 0)),
            pl.BlockSpec((1, 1, BH), lambda i: (i, 0, 0)),
        ],
        out_specs=[
            pl.BlockSpec((NB, 1, BH), lambda i: (0, 0, 0)),
            pl.BlockSpec((TOPK,), lambda i: (0,)),
            pl.BlockSpec((TOPK,), lambda i: (0,)),
        ],
        out_shape=[
            jax.ShapeDtypeStruct((NB, 1, BH), jnp.float32),
            jax.ShapeDtypeStruct((TOPK,), jnp.int32),
            jax.ShapeDtypeStruct((TOPK,), jnp.float32),
        ],
        scratch_shapes=[pltpu.VMEM((NB, BH), jnp.float32)],
        compiler_params=pltpu.CompilerParams(vmem_limit_bytes=100 * 2**20),
    )(emb2, W_enc, benc2)

    cols = _dec_gather()(W_dec, idx, val)

    recon2 = pl.pallas_call(
        _fin_body,
        out_shape=jax.ShapeDtypeStruct((INP // 128, 128), jnp.float32),
    )(cols, b_dec.reshape(INP // 128, 128))

    return enc2.reshape(HID), recon2.reshape(INP)
